# baseline (speedup 1.0000x reference)
import jax
import jax.numpy as jnp
from jax.experimental import pallas as pl
from jax.experimental.pallas import tpu as pltpu

F32 = jnp.float32
T, B, H, E = 30, 16, 256, 256
NPU, L, O = 8, 10, 16
P = 32 * 20 * 20


def _sig(x):
    return jax.nn.sigmoid(x)


def _lstm0_body(x_ref, wih_ref, whh_ref, b_ref, out_ref, gih_s):
    pid = pl.program_id(0)
    for c in range(5):
        sl = slice(c * 96, (c + 1) * 96)
        gih_s[sl, :] = (
            jnp.dot(x_ref[sl, :], wih_ref[0], preferred_element_type=F32)
            + b_ref[0]
        )

    def step(s, carry):
        h, cc = carry
        t = jnp.where(pid == 0, s, (T - 1) - s)
        base = pl.multiple_of(t * B, B)
        g = gih_s[pl.ds(base, B), :] + jnp.dot(
            h, whh_ref[0], preferred_element_type=F32
        )
        i = _sig(g[:, 0:256])
        f = _sig(g[:, 256:512])
        gg = jnp.tanh(g[:, 512:768])
        o = _sig(g[:, 768:1024])
        cc = f * cc + i * gg
        h = o * jnp.tanh(cc)
        out_ref[0, pl.ds(base, B), :] = h
        return (h, cc)

    z = jnp.zeros((B, H), F32)
    jax.lax.fori_loop(0, T, step, (z, z))


def _lstm1_body(h0_ref, wih_ref, whh_ref, b_ref, fcT_ref, fcb_ref,
                icap_ref, feats_ref, gih_s, hs_s):
    pid = pl.program_id(0)
    for c in range(5):
        sl = slice(c * 96, (c + 1) * 96)
        gih_s[sl, :] = (
            jnp.dot(h0_ref[0, sl, :], wih_ref[0, 0:256, :],
                    preferred_element_type=F32)
            + jnp.dot(h0_ref[1, sl, :], wih_ref[0, 256:512, :],
                      preferred_element_type=F32)
            + b_ref[0]
        )

    def step(s, carry):
        h, cc, hsum = carry
        t = jnp.where(pid == 0, s, (T - 1) - s)
        base = pl.multiple_of(t * B, B)
        g = gih_s[pl.ds(base, B), :] + jnp.dot(
            h, whh_ref[0], preferred_element_type=F32
        )
        i = _sig(g[:, 0:256])
        f = _sig(g[:, 256:512])
        gg = jnp.tanh(g[:, 512:768])
        o = _sig(g[:, 768:1024])
        cc = f * cc + i * gg
        h = o * jnp.tanh(cc)
        hs_s[pl.ds(base, B), :] = h
        return (h, cc, hsum + h)

    z = jnp.zeros((B, H), F32)
    _, _, hsum = jax.lax.fori_loop(0, T, step, (z, z, z))
    feats_ref[0] = hsum * (1.0 / T)
    wsel = jnp.where(pid == 0, 1.0, 0.0)
    for c in range(5):
        sl = slice(c * 96, (c + 1) * 96)
        icap_ref[0, sl, :] = (
            jnp.dot(hs_s[sl, :], fcT_ref[0], preferred_element_type=F32)
            + wsel * fcb_ref[...]
        )


def _conv_body(x_ref, w1_ref, b1_ref, wk_ref, b2_ref, out_ref, x1_s):
    for c in range(6):
        sl = slice(c * 96, (c + 1) * 96)
        x1_s[sl, :] = jax.nn.relu(
            jnp.dot(x_ref[0, sl, :], w1_ref[...], preferred_element_type=F32)
            + b1_ref[...]
        )
    x1_s[576:584, :] = jnp.zeros((8, 256), F32)
    for c in range(4):
        accs = [jnp.zeros((120, 256), F32) for _ in range(3)]
        for k in range(25):
            off = (k // 5) * 24 + (k % 5) + c * 120
            accs[k % 3] = accs[k % 3] + jnp.dot(
                x1_s[off:off + 120, :], wk_ref[k],
                preferred_element_type=F32,
            )
        out_ref[0, c * 120:(c + 1) * 120, :] = (
            accs[0] + accs[1] + accs[2] + b2_ref[...]
        )


def _dcap_body(u_ref, w_ref, out_ref):
    msq = jnp.zeros((B, 1), F32)
    acc = jnp.zeros((B, L * O), F32)
    for c in range(10):
        blk = u_ref[0, :, c * 1280:(c + 1) * 1280]
        msq = msq + jnp.sum(blk * blk, axis=1, keepdims=True)
        acc = acc + jnp.dot(
            blk, w_ref[0, c * 1280:(c + 1) * 1280, :],
            preferred_element_type=F32,
        )
    scale = jnp.sqrt(msq) / (1.0 + msq)
    out_ref[0] = acc * scale


def _final_body(p_ref, out_ref):
    s = jnp.sum(p_ref[...], axis=0)
    msq = jnp.zeros((B, O), F32)
    for l in range(L):
        sl = s[:, l * O:(l + 1) * O]
        msq = msq + sl * sl
    fac = jnp.sqrt(msq) / (1.0 + msq)
    out_ref[...] = s * jnp.concatenate([fac] * L, axis=1)


def kernel(input, embed, lstm0_Wih, lstm0_Whh, lstm0_bih, lstm0_bhh,
           lstm1_Wih, lstm1_Whh, lstm1_bih, lstm1_bhh,
           fc_W, fc_b, conv1_W, conv1_b, pconv_W, pconv_b, dcap_W):
    ids = input.astype(jnp.int32)
    enc = jnp.take(embed, ids, axis=0)
    x_tm = enc.transpose(1, 0, 2).reshape(T * B, E)

    wih0 = jnp.swapaxes(lstm0_Wih, 1, 2)
    whh0 = jnp.swapaxes(lstm0_Whh, 1, 2)
    b0 = (lstm0_bih + lstm0_bhh)[:, None, :]
    wih1 = jnp.swapaxes(lstm1_Wih, 1, 2)
    whh1 = jnp.swapaxes(lstm1_Whh, 1, 2)
    b1l = (lstm1_bih + lstm1_bhh)[:, None, :]
    fcT = jnp.stack([fc_W[:, :H].T, fc_W[:, H:].T])
    fcb = fc_b[None, :]

    h0 = pl.pallas_call(
        _lstm0_body,
        grid=(2,),
        in_specs=[
            pl.BlockSpec((T * B, E), lambda d: (0, 0)),
            pl.BlockSpec((1, E, 4 * H), lambda d: (d, 0, 0)),
            pl.BlockSpec((1, H, 4 * H), lambda d: (d, 0, 0)),
            pl.BlockSpec((1, 1, 4 * H), lambda d: (d, 0, 0)),
        ],
        out_specs=pl.BlockSpec((1, T * B, H), lambda d: (d, 0, 0)),
        out_shape=jax.ShapeDtypeStruct((2, T * B, H), F32),
        scratch_shapes=[pltpu.VMEM((T * B, 4 * H), F32)],
        compiler_params=pltpu.CompilerParams(
            dimension_semantics=("parallel",)),
    )(x_tm, wih0, whh0, b0)

    icap_p, feats_p = pl.pallas_call(
        _lstm1_body,
        grid=(2,),
        in_specs=[
            pl.BlockSpec((2, T * B, H), lambda d: (0, 0, 0)),
            pl.BlockSpec((1, 2 * H, 4 * H), lambda d: (d, 0, 0)),
            pl.BlockSpec((1, H, 4 * H), lambda d: (d, 0, 0)),
            pl.BlockSpec((1, 1, 4 * H), lambda d: (d, 0, 0)),
            pl.BlockSpec((1, H, T), lambda d: (d, 0, 0)),
            pl.BlockSpec((1, T), lambda d: (0, 0)),
        ],
        out_specs=[
            pl.BlockSpec((1, T * B, T), lambda d: (d, 0, 0)),
            pl.BlockSpec((1, B, H), lambda d: (d, 0, 0)),
        ],
        out_shape=[
            jax.ShapeDtypeStruct((2, T * B, T), F32),
            jax.ShapeDtypeStruct((2, B, H), F32),
        ],
        scratch_shapes=[pltpu.VMEM((T * B, 4 * H), F32),
                        pltpu.VMEM((T * B, H), F32)],
        compiler_params=pltpu.CompilerParams(
            dimension_semantics=("parallel",)),
    )(h0, wih1, whh1, b1l, fcT, fcb)

    lstm_feats = jnp.concatenate([feats_p[0], feats_p[1]], axis=-1)

    ic = (icap_p[0] + icap_p[1]).reshape(T, B, T).transpose(1, 0, 2)
    pat = jnp.stack(
        [ic[:, di:di + 24, dj:dj + 24] for di in range(7) for dj in range(7)],
        axis=-1,
    ).reshape(B, 576, 49)
    pat = jnp.pad(pat, ((0, 0), (0, 0), (0, 79)))
    w1 = jnp.pad(conv1_W.reshape(256, 49).T, ((0, 79), (0, 0)))
    b1c = conv1_b[None, :]
    wk = pconv_W.reshape(256, 256, 5, 5).transpose(2, 3, 1, 0).reshape(
        25, 256, 256)
    b2c = pconv_b.reshape(1, 256)

    yw = pl.pallas_call(
        _conv_body,
        grid=(B,),
        in_specs=[
            pl.BlockSpec((1, 576, 128), lambda d: (d, 0, 0)),
            pl.BlockSpec((128, 256), lambda d: (0, 0)),
            pl.BlockSpec((1, 256), lambda d: (0, 0)),
            pl.BlockSpec((25, 256, 256), lambda d: (0, 0, 0)),
            pl.BlockSpec((1, 256), lambda d: (0, 0)),
        ],
        out_specs=pl.BlockSpec((1, 480, 256), lambda d: (d, 0, 0)),
        out_shape=jax.ShapeDtypeStruct((B, 480, 256), F32),
        scratch_shapes=[pltpu.VMEM((584, 256), F32)],
        compiler_params=pltpu.CompilerParams(
            dimension_semantics=("parallel",)),
    )(pat, w1, b1c, wk, b2c)

    yv = yw.reshape(B, 20, 24, 256)[:, :, :20, :]
    units = (yv.transpose(3, 0, 1, 2).reshape(NPU, 32, B, 400)
             .transpose(0, 2, 1, 3).reshape(NPU, B, P))
    wn = dcap_W[0].transpose(3, 0, 1, 2).reshape(NPU, P, L * O)

    parts = pl.pallas_call(
        _dcap_body,
        grid=(NPU,),
        in_specs=[
            pl.BlockSpec((1, B, P), lambda d: (d, 0, 0)),
            pl.BlockSpec((1, P, L * O), lambda d: (d, 0, 0)),
        ],
        out_specs=pl.BlockSpec((1, B, L * O), lambda d: (d, 0, 0)),
        out_shape=jax.ShapeDtypeStruct((NPU, B, L * O), F32),
        compiler_params=pltpu.CompilerParams(
            dimension_semantics=("parallel",)),
    )(units, wn)

    v = pl.pallas_call(
        _final_body,
        out_shape=jax.ShapeDtypeStruct((B, L * O), F32),
    )(parts)

    return (v.reshape(B, L, O), lstm_feats)


# speedup vs baseline: 1.1552x; 1.1552x over previous
import jax
import jax.numpy as jnp
from jax.experimental import pallas as pl
from jax.experimental.pallas import tpu as pltpu

F32 = jnp.float32
T, B, H, E = 30, 16, 256, 256
NPU, L, O = 8, 10, 16
P = 32 * 20 * 20


def _sig(x):
    return jax.nn.sigmoid(x)


def _lstm0_body(x_ref, wih_ref, whh_ref, b_ref, out_ref, gih_s):
    pid = pl.program_id(0)
    for c in range(5):
        sl = slice(c * 96, (c + 1) * 96)
        gih_s[sl, :] = (
            jnp.dot(x_ref[sl, :], wih_ref[0], preferred_element_type=F32)
            + b_ref[0]
        )

    def step(s, carry):
        h, cc = carry
        t = jnp.where(pid == 0, s, (T - 1) - s)
        base = pl.multiple_of(t * B, B)
        g = gih_s[pl.ds(base, B), :] + jnp.dot(
            h, whh_ref[0], preferred_element_type=F32
        )
        i = _sig(g[:, 0:256])
        f = _sig(g[:, 256:512])
        gg = jnp.tanh(g[:, 512:768])
        o = _sig(g[:, 768:1024])
        cc = f * cc + i * gg
        h = o * jnp.tanh(cc)
        out_ref[0, pl.ds(base, B), :] = h
        return (h, cc)

    z = jnp.zeros((B, H), F32)
    jax.lax.fori_loop(0, T, step, (z, z))


def _lstm1_body(h0_ref, wih_ref, whh_ref, b_ref, fcT_ref, fcb_ref,
                icap_ref, feats_ref, gih_s, hs_s):
    pid = pl.program_id(0)
    for c in range(5):
        sl = slice(c * 96, (c + 1) * 96)
        gih_s[sl, :] = (
            jnp.dot(h0_ref[0, sl, :], wih_ref[0, 0:256, :],
                    preferred_element_type=F32)
            + jnp.dot(h0_ref[1, sl, :], wih_ref[0, 256:512, :],
                      preferred_element_type=F32)
            + b_ref[0]
        )

    def step(s, carry):
        h, cc, hsum = carry
        t = jnp.where(pid == 0, s, (T - 1) - s)
        base = pl.multiple_of(t * B, B)
        g = gih_s[pl.ds(base, B), :] + jnp.dot(
            h, whh_ref[0], preferred_element_type=F32
        )
        i = _sig(g[:, 0:256])
        f = _sig(g[:, 256:512])
        gg = jnp.tanh(g[:, 512:768])
        o = _sig(g[:, 768:1024])
        cc = f * cc + i * gg
        h = o * jnp.tanh(cc)
        hs_s[pl.ds(base, B), :] = h
        return (h, cc, hsum + h)

    z = jnp.zeros((B, H), F32)
    _, _, hsum = jax.lax.fori_loop(0, T, step, (z, z, z))
    feats_ref[0] = hsum * (1.0 / T)
    wsel = jnp.where(pid == 0, 1.0, 0.0)
    for c in range(5):
        sl = slice(c * 96, (c + 1) * 96)
        icap_ref[0, sl, :] = (
            jnp.dot(hs_s[sl, :], fcT_ref[0], preferred_element_type=F32)
            + wsel * fcb_ref[...]
        )


def _conv_body(x_ref, w1_ref, b1_ref, wk_ref, b2_ref, out_ref, x1_s):
    for c in range(6):
        sl = slice(c * 96, (c + 1) * 96)
        x1_s[sl, :] = jax.nn.relu(
            jnp.dot(x_ref[0, sl, :], w1_ref[...], preferred_element_type=F32)
            + b1_ref[...]
        ).astype(jnp.bfloat16)
    x1_s[576:584, :] = jnp.zeros((8, 256), jnp.bfloat16)
    for c in range(4):
        accs = [jnp.zeros((120, 256), F32) for _ in range(3)]
        for k in range(25):
            off = (k // 5) * 24 + (k % 5) + c * 120
            accs[k % 3] = accs[k % 3] + jnp.dot(
                x1_s[off:off + 120, :], wk_ref[k],
                preferred_element_type=F32,
            )
        out_ref[0, c * 120:(c + 1) * 120, :] = (
            accs[0] + accs[1] + accs[2] + b2_ref[...]
        )


def _dcap_body(u_ref, w_ref, out_ref):
    msq = jnp.zeros((B, 1), F32)
    acc = jnp.zeros((B, L * O), F32)
    for c in range(10):
        blk = u_ref[0, :, c * 1280:(c + 1) * 1280]
        msq = msq + jnp.sum(blk * blk, axis=1, keepdims=True)
        acc = acc + jnp.dot(
            blk.astype(jnp.bfloat16), w_ref[0, c * 1280:(c + 1) * 1280, :],
            preferred_element_type=F32,
        )
    scale = jnp.sqrt(msq) / (1.0 + msq)
    out_ref[0] = acc * scale


def _final_body(p_ref, out_ref):
    s = jnp.sum(p_ref[...], axis=0)
    msq = jnp.zeros((B, O), F32)
    for l in range(L):
        sl = s[:, l * O:(l + 1) * O]
        msq = msq + sl * sl
    fac = jnp.sqrt(msq) / (1.0 + msq)
    out_ref[...] = s * jnp.concatenate([fac] * L, axis=1)


def kernel(input, embed, lstm0_Wih, lstm0_Whh, lstm0_bih, lstm0_bhh,
           lstm1_Wih, lstm1_Whh, lstm1_bih, lstm1_bhh,
           fc_W, fc_b, conv1_W, conv1_b, pconv_W, pconv_b, dcap_W):
    ids = input.astype(jnp.int32)
    enc = jnp.take(embed, ids, axis=0)
    x_tm = enc.transpose(1, 0, 2).reshape(T * B, E)

    wih0 = jnp.swapaxes(lstm0_Wih, 1, 2)
    whh0 = jnp.swapaxes(lstm0_Whh, 1, 2)
    b0 = (lstm0_bih + lstm0_bhh)[:, None, :]
    wih1 = jnp.swapaxes(lstm1_Wih, 1, 2)
    whh1 = jnp.swapaxes(lstm1_Whh, 1, 2)
    b1l = (lstm1_bih + lstm1_bhh)[:, None, :]
    fcT = jnp.stack([fc_W[:, :H].T, fc_W[:, H:].T])
    fcb = fc_b[None, :]

    h0 = pl.pallas_call(
        _lstm0_body,
        grid=(2,),
        in_specs=[
            pl.BlockSpec((T * B, E), lambda d: (0, 0)),
            pl.BlockSpec((1, E, 4 * H), lambda d: (d, 0, 0)),
            pl.BlockSpec((1, H, 4 * H), lambda d: (d, 0, 0)),
            pl.BlockSpec((1, 1, 4 * H), lambda d: (d, 0, 0)),
        ],
        out_specs=pl.BlockSpec((1, T * B, H), lambda d: (d, 0, 0)),
        out_shape=jax.ShapeDtypeStruct((2, T * B, H), F32),
        scratch_shapes=[pltpu.VMEM((T * B, 4 * H), F32)],
        compiler_params=pltpu.CompilerParams(
            dimension_semantics=("parallel",)),
    )(x_tm, wih0, whh0, b0)

    icap_p, feats_p = pl.pallas_call(
        _lstm1_body,
        grid=(2,),
        in_specs=[
            pl.BlockSpec((2, T * B, H), lambda d: (0, 0, 0)),
            pl.BlockSpec((1, 2 * H, 4 * H), lambda d: (d, 0, 0)),
            pl.BlockSpec((1, H, 4 * H), lambda d: (d, 0, 0)),
            pl.BlockSpec((1, 1, 4 * H), lambda d: (d, 0, 0)),
            pl.BlockSpec((1, H, T), lambda d: (d, 0, 0)),
            pl.BlockSpec((1, T), lambda d: (0, 0)),
        ],
        out_specs=[
            pl.BlockSpec((1, T * B, T), lambda d: (d, 0, 0)),
            pl.BlockSpec((1, B, H), lambda d: (d, 0, 0)),
        ],
        out_shape=[
            jax.ShapeDtypeStruct((2, T * B, T), F32),
            jax.ShapeDtypeStruct((2, B, H), F32),
        ],
        scratch_shapes=[pltpu.VMEM((T * B, 4 * H), F32),
                        pltpu.VMEM((T * B, H), F32)],
        compiler_params=pltpu.CompilerParams(
            dimension_semantics=("parallel",)),
    )(h0, wih1, whh1, b1l, fcT, fcb)

    lstm_feats = jnp.concatenate([feats_p[0], feats_p[1]], axis=-1)

    ic = (icap_p[0] + icap_p[1]).reshape(T, B, T).transpose(1, 0, 2)
    pat = jnp.stack(
        [ic[:, di:di + 24, dj:dj + 24] for di in range(7) for dj in range(7)],
        axis=-1,
    ).reshape(B, 576, 49)
    pat = jnp.pad(pat, ((0, 0), (0, 0), (0, 79)))
    pat = pat.astype(jnp.bfloat16)
    w1 = jnp.pad(conv1_W.reshape(256, 49).T,
                 ((0, 79), (0, 0))).astype(jnp.bfloat16)
    b1c = conv1_b[None, :]
    wk = pconv_W.reshape(256, 256, 5, 5).transpose(2, 3, 1, 0).reshape(
        25, 256, 256).astype(jnp.bfloat16)
    b2c = pconv_b.reshape(1, 256)

    yw = pl.pallas_call(
        _conv_body,
        grid=(B,),
        in_specs=[
            pl.BlockSpec((1, 576, 128), lambda d: (d, 0, 0)),
            pl.BlockSpec((128, 256), lambda d: (0, 0)),
            pl.BlockSpec((1, 256), lambda d: (0, 0)),
            pl.BlockSpec((25, 256, 256), lambda d: (0, 0, 0)),
            pl.BlockSpec((1, 256), lambda d: (0, 0)),
        ],
        out_specs=pl.BlockSpec((1, 480, 256), lambda d: (d, 0, 0)),
        out_shape=jax.ShapeDtypeStruct((B, 480, 256), F32),
        scratch_shapes=[pltpu.VMEM((584, 256), jnp.bfloat16)],
        compiler_params=pltpu.CompilerParams(
            dimension_semantics=("parallel",)),
    )(pat, w1, b1c, wk, b2c)

    yv = yw.reshape(B, 20, 24, 256)[:, :, :20, :]
    units = (yv.transpose(3, 0, 1, 2).reshape(NPU, 32, B, 400)
             .transpose(0, 2, 1, 3).reshape(NPU, B, P))
    wn = dcap_W[0].transpose(3, 0, 1, 2).reshape(
        NPU, P, L * O).astype(jnp.bfloat16)

    parts = pl.pallas_call(
        _dcap_body,
        grid=(NPU,),
        in_specs=[
            pl.BlockSpec((1, B, P), lambda d: (d, 0, 0)),
            pl.BlockSpec((1, P, L * O), lambda d: (d, 0, 0)),
        ],
        out_specs=pl.BlockSpec((1, B, L * O), lambda d: (d, 0, 0)),
        out_shape=jax.ShapeDtypeStruct((NPU, B, L * O), F32),
        compiler_params=pltpu.CompilerParams(
            dimension_semantics=("parallel",)),
    )(units, wn)

    v = pl.pallas_call(
        _final_body,
        out_shape=jax.ShapeDtypeStruct((B, L * O), F32),
    )(parts)

    return (v.reshape(B, L, O), lstm_feats)


# speedup vs baseline: 1.2389x; 1.0725x over previous
import jax
import jax.numpy as jnp
from jax.experimental import pallas as pl
from jax.experimental.pallas import tpu as pltpu

F32 = jnp.float32
T, B, H, E = 30, 16, 256, 256
NPU, L, O = 8, 10, 16
P = 32 * 20 * 20


def _sig(x):
    return jax.nn.sigmoid(x)


def _lstm0_body(x_ref, wih_ref, whh_ref, b_ref, out_ref, gih_s):
    pid = pl.program_id(0)
    for c in range(5):
        sl = slice(c * 96, (c + 1) * 96)
        gih_s[sl, :] = (
            jnp.dot(x_ref[sl, :], wih_ref[0], preferred_element_type=F32)
            + b_ref[0]
        )

    def step(s, carry):
        h, cc = carry
        t = jnp.where(pid == 0, s, (T - 1) - s)
        base = pl.multiple_of(t * B, B)
        g = gih_s[pl.ds(base, B), :] + jnp.dot(
            h.astype(jnp.bfloat16), whh_ref[0], preferred_element_type=F32
        )
        i = _sig(g[:, 0:256])
        f = _sig(g[:, 256:512])
        gg = jnp.tanh(g[:, 512:768])
        o = _sig(g[:, 768:1024])
        cc = f * cc + i * gg
        h = o * jnp.tanh(cc)
        out_ref[0, pl.ds(base, B), :] = h.astype(jnp.bfloat16)
        return (h, cc)

    z = jnp.zeros((B, H), F32)
    jax.lax.fori_loop(0, T, step, (z, z))


def _lstm1_body(h0_ref, wih_ref, whh_ref, b_ref, fcT_ref, fcb_ref,
                icap_ref, feats_ref, gih_s, hs_s):
    pid = pl.program_id(0)
    for c in range(5):
        sl = slice(c * 96, (c + 1) * 96)
        gih_s[sl, :] = (
            jnp.dot(h0_ref[0, sl, :], wih_ref[0, 0:256, :],
                    preferred_element_type=F32)
            + jnp.dot(h0_ref[1, sl, :], wih_ref[0, 256:512, :],
                      preferred_element_type=F32)
            + b_ref[0]
        )

    def step(s, carry):
        h, cc, hsum = carry
        t = jnp.where(pid == 0, s, (T - 1) - s)
        base = pl.multiple_of(t * B, B)
        g = gih_s[pl.ds(base, B), :] + jnp.dot(
            h.astype(jnp.bfloat16), whh_ref[0], preferred_element_type=F32
        )
        i = _sig(g[:, 0:256])
        f = _sig(g[:, 256:512])
        gg = jnp.tanh(g[:, 512:768])
        o = _sig(g[:, 768:1024])
        cc = f * cc + i * gg
        h = o * jnp.tanh(cc)
        hs_s[pl.ds(base, B), :] = h.astype(jnp.bfloat16)
        return (h, cc, hsum + h)

    z = jnp.zeros((B, H), F32)
    _, _, hsum = jax.lax.fori_loop(0, T, step, (z, z, z))
    feats_ref[0] = hsum * (1.0 / T)
    wsel = jnp.where(pid == 0, 1.0, 0.0)
    for c in range(5):
        sl = slice(c * 96, (c + 1) * 96)
        icap_ref[0, sl, :] = (
            jnp.dot(hs_s[sl, :], fcT_ref[0], preferred_element_type=F32)
            + wsel * fcb_ref[...]
        )


def _conv_body(x_ref, w1_ref, b1_ref, wk_ref, b2_ref, out_ref, x1_s):
    for c in range(6):
        sl = slice(c * 96, (c + 1) * 96)
        x1_s[sl, :] = jax.nn.relu(
            jnp.dot(x_ref[0, sl, :], w1_ref[...], preferred_element_type=F32)
            + b1_ref[...]
        ).astype(jnp.bfloat16)
    x1_s[576:584, :] = jnp.zeros((8, 256), jnp.bfloat16)
    for c in range(4):
        accs = [jnp.zeros((120, 256), F32) for _ in range(3)]
        for k in range(25):
            off = (k // 5) * 24 + (k % 5) + c * 120
            accs[k % 3] = accs[k % 3] + jnp.dot(
                x1_s[off:off + 120, :], wk_ref[k],
                preferred_element_type=F32,
            )
        out_ref[0, c * 120:(c + 1) * 120, :] = (
            accs[0] + accs[1] + accs[2] + b2_ref[...]
        )


def _dcap_body(u_ref, w_ref, out_ref):
    msq = jnp.zeros((B, 1), F32)
    acc = jnp.zeros((B, L * O), F32)
    for c in range(10):
        blk = u_ref[0, :, c * 1280:(c + 1) * 1280]
        msq = msq + jnp.sum(blk * blk, axis=1, keepdims=True)
        acc = acc + jnp.dot(
            blk.astype(jnp.bfloat16), w_ref[0, c * 1280:(c + 1) * 1280, :],
            preferred_element_type=F32,
        )
    scale = jnp.sqrt(msq) / (1.0 + msq)
    out_ref[0] = acc * scale


def _final_body(p_ref, out_ref):
    s = jnp.sum(p_ref[...], axis=0)
    msq = jnp.zeros((B, O), F32)
    for l in range(L):
        sl = s[:, l * O:(l + 1) * O]
        msq = msq + sl * sl
    fac = jnp.sqrt(msq) / (1.0 + msq)
    out_ref[...] = s * jnp.concatenate([fac] * L, axis=1)


def kernel(input, embed, lstm0_Wih, lstm0_Whh, lstm0_bih, lstm0_bhh,
           lstm1_Wih, lstm1_Whh, lstm1_bih, lstm1_bhh,
           fc_W, fc_b, conv1_W, conv1_b, pconv_W, pconv_b, dcap_W):
    ids = input.astype(jnp.int32)
    enc = jnp.take(embed, ids, axis=0)
    x_tm = enc.transpose(1, 0, 2).reshape(T * B, E).astype(jnp.bfloat16)

    bf = jnp.bfloat16
    wih0 = jnp.swapaxes(lstm0_Wih, 1, 2).astype(bf)
    whh0 = jnp.swapaxes(lstm0_Whh, 1, 2).astype(bf)
    b0 = (lstm0_bih + lstm0_bhh)[:, None, :]
    wih1 = jnp.swapaxes(lstm1_Wih, 1, 2).astype(bf)
    whh1 = jnp.swapaxes(lstm1_Whh, 1, 2).astype(bf)
    b1l = (lstm1_bih + lstm1_bhh)[:, None, :]
    fcT = jnp.stack([fc_W[:, :H].T, fc_W[:, H:].T]).astype(bf)
    fcb = fc_b[None, :]

    h0 = pl.pallas_call(
        _lstm0_body,
        grid=(2,),
        in_specs=[
            pl.BlockSpec((T * B, E), lambda d: (0, 0)),
            pl.BlockSpec((1, E, 4 * H), lambda d: (d, 0, 0)),
            pl.BlockSpec((1, H, 4 * H), lambda d: (d, 0, 0)),
            pl.BlockSpec((1, 1, 4 * H), lambda d: (d, 0, 0)),
        ],
        out_specs=pl.BlockSpec((1, T * B, H), lambda d: (d, 0, 0)),
        out_shape=jax.ShapeDtypeStruct((2, T * B, H), jnp.bfloat16),
        scratch_shapes=[pltpu.VMEM((T * B, 4 * H), F32)],
        compiler_params=pltpu.CompilerParams(
            dimension_semantics=("parallel",)),
    )(x_tm, wih0, whh0, b0)

    icap_p, feats_p = pl.pallas_call(
        _lstm1_body,
        grid=(2,),
        in_specs=[
            pl.BlockSpec((2, T * B, H), lambda d: (0, 0, 0)),
            pl.BlockSpec((1, 2 * H, 4 * H), lambda d: (d, 0, 0)),
            pl.BlockSpec((1, H, 4 * H), lambda d: (d, 0, 0)),
            pl.BlockSpec((1, 1, 4 * H), lambda d: (d, 0, 0)),
            pl.BlockSpec((1, H, T), lambda d: (d, 0, 0)),
            pl.BlockSpec((1, T), lambda d: (0, 0)),
        ],
        out_specs=[
            pl.BlockSpec((1, T * B, T), lambda d: (d, 0, 0)),
            pl.BlockSpec((1, B, H), lambda d: (d, 0, 0)),
        ],
        out_shape=[
            jax.ShapeDtypeStruct((2, T * B, T), F32),
            jax.ShapeDtypeStruct((2, B, H), F32),
        ],
        scratch_shapes=[pltpu.VMEM((T * B, 4 * H), F32),
                        pltpu.VMEM((T * B, H), jnp.bfloat16)],
        compiler_params=pltpu.CompilerParams(
            dimension_semantics=("parallel",)),
    )(h0, wih1, whh1, b1l, fcT, fcb)

    lstm_feats = jnp.concatenate([feats_p[0], feats_p[1]], axis=-1)

    ic = (icap_p[0] + icap_p[1]).reshape(T, B, T).transpose(1, 0, 2)
    pat = jnp.stack(
        [ic[:, di:di + 24, dj:dj + 24] for di in range(7) for dj in range(7)],
        axis=-1,
    ).reshape(B, 576, 49)
    pat = jnp.pad(pat, ((0, 0), (0, 0), (0, 79)))
    pat = pat.astype(jnp.bfloat16)
    w1 = jnp.pad(conv1_W.reshape(256, 49).T,
                 ((0, 79), (0, 0))).astype(jnp.bfloat16)
    b1c = conv1_b[None, :]
    wk = pconv_W.reshape(256, 256, 5, 5).transpose(2, 3, 1, 0).reshape(
        25, 256, 256).astype(jnp.bfloat16)
    b2c = pconv_b.reshape(1, 256)

    yw = pl.pallas_call(
        _conv_body,
        grid=(B,),
        in_specs=[
            pl.BlockSpec((1, 576, 128), lambda d: (d, 0, 0)),
            pl.BlockSpec((128, 256), lambda d: (0, 0)),
            pl.BlockSpec((1, 256), lambda d: (0, 0)),
            pl.BlockSpec((25, 256, 256), lambda d: (0, 0, 0)),
            pl.BlockSpec((1, 256), lambda d: (0, 0)),
        ],
        out_specs=pl.BlockSpec((1, 480, 256), lambda d: (d, 0, 0)),
        out_shape=jax.ShapeDtypeStruct((B, 480, 256), F32),
        scratch_shapes=[pltpu.VMEM((584, 256), jnp.bfloat16)],
        compiler_params=pltpu.CompilerParams(
            dimension_semantics=("parallel",)),
    )(pat, w1, b1c, wk, b2c)

    yv = yw.reshape(B, 20, 24, 256)[:, :, :20, :]
    units = (yv.transpose(3, 0, 1, 2).reshape(NPU, 32, B, 400)
             .transpose(0, 2, 1, 3).reshape(NPU, B, P))
    wn = dcap_W[0].transpose(3, 0, 1, 2).reshape(
        NPU, P, L * O).astype(jnp.bfloat16)

    parts = pl.pallas_call(
        _dcap_body,
        grid=(NPU,),
        in_specs=[
            pl.BlockSpec((1, B, P), lambda d: (d, 0, 0)),
            pl.BlockSpec((1, P, L * O), lambda d: (d, 0, 0)),
        ],
        out_specs=pl.BlockSpec((1, B, L * O), lambda d: (d, 0, 0)),
        out_shape=jax.ShapeDtypeStruct((NPU, B, L * O), F32),
        compiler_params=pltpu.CompilerParams(
            dimension_semantics=("parallel",)),
    )(units, wn)

    v = pl.pallas_call(
        _final_body,
        out_shape=jax.ShapeDtypeStruct((B, L * O), F32),
    )(parts)

    return (v.reshape(B, L, O), lstm_feats)
